# baseline (speedup 1.0000x reference)
import functools
import math

import jax
import jax.numpy as jnp
from jax import lax
from jax.experimental import pallas as pl
from jax.experimental.pallas import tpu as pltpu
from jax.experimental.pallas import tpu_sc as plsc

N_NODES = 10000
NC = 2
NS = 16
NW = NC * NS
SUB = 128
NBUF = 8
FS = 8
K0_FRAC = 0.5

N_PAD = ((N_NODES + 1 + NS * 8 - 1) // (NS * 8)) * (NS * 8)
ROWS_PT = N_PAD // NS
PADR = N_PAD - N_NODES


def _make_agg(F, K):
  mesh = plsc.VectorSubcoreMesh(
      core_axis_name="c", subcore_axis_name="s", num_cores=NC, num_subcores=NS)

  def body(y_hbm, src_hbm, dst_hbm, zeros_hbm, out_hbm, idx_s, idx_d, rows,
           acc, gsem, ssem):
    K0, K1 = K
    c = lax.axis_index("c")
    s = lax.axis_index("s")
    wid = c * NS + s
    pltpu.sync_copy(zeros_hbm.at[s], acc.at[pl.ds(s * ROWS_PT, ROWS_PT)])
    pltpu.sync_copy(src_hbm.at[wid], idx_s)
    pltpu.sync_copy(dst_hbm.at[wid], idx_d)
    plsc.subcore_barrier()
    kc = jnp.where(c == 0, K0, K1)

    @pl.loop(0, kc, step=NBUF)
    def _chunk_group(j0):
      gets = [
          pltpu.async_copy(y_hbm.at[idx_s.at[j0 + b]], rows.at[b], gsem)
          for b in range(NBUF)
      ]
      for cp in gets:
        cp.wait()
      puts = [
          pltpu.async_copy(rows.at[b], acc.at[idx_d.at[j0 + b]], ssem,
                           add=True)
          for b in range(NBUF)
      ]
      for cp in puts:
        cp.wait()

    plsc.subcore_barrier()
    pltpu.sync_copy(acc.at[pl.ds(s * ROWS_PT, ROWS_PT)], out_hbm.at[c, s])

  return pl.kernel(
      body,
      out_type=jax.ShapeDtypeStruct((NC, NS, ROWS_PT, F), jnp.float32),
      mesh=mesh,
      compiler_params=pltpu.CompilerParams(use_tc_tiling_on_sc=False),
      scratch_types=[
          pltpu.VMEM((max(K), SUB), jnp.int32),
          pltpu.VMEM((max(K), SUB), jnp.int32),
          pltpu.VMEM((NBUF, SUB, F), jnp.float32),
          pltpu.VMEM_SHARED((N_PAD, F), jnp.float32),
          pltpu.SemaphoreType.DMA,
          pltpu.SemaphoreType.DMA,
      ],
  )


def _stage1_body(degp_ref, x_ref, w1_ref, dis_ref, y1_ref):
  deg = degp_ref[0, :N_NODES, :1] + degp_ref[1, :N_NODES, :1] + 1.0
  dis = lax.rsqrt(deg)
  dis_ref[...] = dis
  xw = jnp.dot(x_ref[...], w1_ref[...], preferred_element_type=jnp.float32)
  y1_ref[:N_NODES, :] = xw * dis
  y1_ref[pl.ds(N_NODES, PADR), :] = jnp.zeros((PADR, 32), jnp.float32)


def _stage_mid_body(aggp_ref, y_ref, dis_ref, b_ref, w_ref, ynext_ref):
  agg = (aggp_ref[0, :N_NODES, :] + aggp_ref[1, :N_NODES, :]
         + y_ref[:N_NODES, :])
  dis = dis_ref[...]
  h = jnp.maximum(dis * agg + b_ref[...], 0.0)
  ynext_ref[:N_NODES, :] = dis * jnp.dot(
      h, w_ref[...], preferred_element_type=jnp.float32)
  ynext_ref[pl.ds(N_NODES, PADR), :] = jnp.zeros(
      (PADR, ynext_ref.shape[1]), jnp.float32)


def _stage_fin_body(aggp_ref, y_ref, dis_ref, b_ref, out_ref):
  agg = (aggp_ref[0, :N_NODES, :1] + aggp_ref[1, :N_NODES, :1]
         + y_ref[:N_NODES, :1])
  out_ref[...] = dis_ref[...] * agg + b_ref[...]


def _tc(body, out_shapes, *args):
  return pl.pallas_call(body, out_shape=out_shapes)(*args)


def kernel(x, edge_index, W1, b1, W2, b2, W3, b3):
  n = x.shape[0]
  assert n == N_NODES
  src = edge_index[0].astype(jnp.int32)
  dst = edge_index[1].astype(jnp.int32)
  e = src.shape[0]
  ktot = 2 * NBUF * math.ceil(e / (NW * SUB * NBUF))
  k0 = NBUF * round(ktot * K0_FRAC / NBUF)
  k0 = min(max(k0, NBUF), ktot - NBUF)
  k1 = ktot - k0
  kmax = max(k0, k1)
  ep = NS * ktot * SUB
  pad = ep - e
  src = jnp.concatenate([src, jnp.full((pad,), N_NODES, jnp.int32)])
  dst = jnp.concatenate(
      [dst, (jnp.arange(pad, dtype=jnp.int32) * 7) % N_PAD])

  def part(a):
    a0 = a[:NS * k0 * SUB].reshape(NS, k0, SUB)
    a1 = a[NS * k0 * SUB:].reshape(NS, k1, SUB)
    a0 = jnp.pad(a0, ((0, 0), (0, kmax - k0), (0, 0)))
    a1 = jnp.pad(a1, ((0, 0), (0, kmax - k1), (0, 0)))
    return jnp.concatenate([a0, a1], axis=0)

  src3 = part(src)
  dst3 = part(dst)
  k = (k0, k1)

  zeros32 = jnp.zeros((NS, ROWS_PT, 32), jnp.float32)
  zeros8 = jnp.zeros((NS, ROWS_PT, FS), jnp.float32)
  ones8 = jnp.concatenate([
      jnp.ones((N_NODES, FS), jnp.float32),
      jnp.zeros((PADR, FS), jnp.float32)])
  w3p = jnp.pad(W3, ((0, 0), (0, FS - W3.shape[1])))

  agg32 = _make_agg(32, k)
  agg8 = _make_agg(FS, k)

  degp = agg8(ones8, src3, dst3, zeros8).reshape(NC, N_PAD, FS)
  dis, y1 = _tc(
      _stage1_body,
      (jax.ShapeDtypeStruct((N_NODES, 1), jnp.float32),
       jax.ShapeDtypeStruct((N_PAD, 32), jnp.float32)),
      degp, x, W1)

  a1 = agg32(y1, src3, dst3, zeros32).reshape(NC, N_PAD, 32)
  y2 = _tc(_stage_mid_body,
           jax.ShapeDtypeStruct((N_PAD, 32), jnp.float32),
           a1, y1, dis, b1.reshape(1, 32), W2)

  a2 = agg32(y2, src3, dst3, zeros32).reshape(NC, N_PAD, 32)
  y3 = _tc(_stage_mid_body,
           jax.ShapeDtypeStruct((N_PAD, FS), jnp.float32),
           a2, y2, dis, b2.reshape(1, 32), w3p)

  a3 = agg8(y3, src3, dst3, zeros8).reshape(NC, N_PAD, FS)
  out = _tc(_stage_fin_body,
            jax.ShapeDtypeStruct((N_NODES, 1), jnp.float32),
            a3, y3, dis, b3.reshape(1, 1))
  return out[:, 0]


# speedup vs baseline: 1.8605x; 1.8605x over previous
import functools
import math

import jax
import jax.numpy as jnp
from jax import lax
from jax.experimental import pallas as pl
from jax.experimental.pallas import tpu as pltpu
from jax.experimental.pallas import tpu_sc as plsc

N_NODES = 10000
NC = 2
NS = 16
NW = NC * NS
SUB = 128
NBUF = 8
FS = 8
K0_FRAC = 0.5

N_PAD = ((N_NODES + 1 + NS * 8 - 1) // (NS * 8)) * (NS * 8)
ROWS_PT = N_PAD // NS
PADR = N_PAD - N_NODES


def _make_agg(F, K):
  mesh = plsc.VectorSubcoreMesh(
      core_axis_name="c", subcore_axis_name="s", num_cores=NC, num_subcores=NS)

  def body(y_hbm, src_hbm, dst_hbm, zeros_hbm, out_hbm, idx_s, idx_d, rows,
           acc, gsem, ssem):
    K0, K1 = K
    c = lax.axis_index("c")
    s = lax.axis_index("s")
    wid = c * NS + s
    pltpu.sync_copy(zeros_hbm.at[s], acc.at[pl.ds(s * ROWS_PT, ROWS_PT)])
    pltpu.sync_copy(src_hbm.at[wid], idx_s)
    pltpu.sync_copy(dst_hbm.at[wid], idx_d)
    plsc.subcore_barrier()
    kc = jnp.where(c == 0, K0, K1)

    @pl.loop(0, kc, step=NBUF)
    def _chunk_group(j0):
      gets = [
          pltpu.async_copy(y_hbm.at[idx_s.at[j0 + b]], rows.at[b], gsem)
          for b in range(NBUF)
      ]
      for cp in gets:
        cp.wait()
      puts = [
          pltpu.async_copy(rows.at[b], acc.at[idx_d.at[j0 + b]], ssem,
                           add=True)
          for b in range(NBUF)
      ]
      for cp in puts:
        cp.wait()

    plsc.subcore_barrier()
    pltpu.sync_copy(acc.at[pl.ds(s * ROWS_PT, ROWS_PT)], out_hbm.at[c, s])

  return pl.kernel(
      body,
      out_type=jax.ShapeDtypeStruct((NC, NS, ROWS_PT, F), jnp.float32),
      mesh=mesh,
      compiler_params=pltpu.CompilerParams(use_tc_tiling_on_sc=False),
      scratch_types=[
          pltpu.VMEM((max(K), SUB), jnp.int32),
          pltpu.VMEM((max(K), SUB), jnp.int32),
          pltpu.VMEM((NBUF, SUB, F), jnp.float32),
          pltpu.VMEM_SHARED((N_PAD, F), jnp.float32),
          pltpu.SemaphoreType.DMA,
          pltpu.SemaphoreType.DMA,
      ],
  )


def _stage1_body(degp_ref, x_ref, w1_ref, dis_ref, y1_ref):
  deg = degp_ref[0, :N_NODES, :1] + degp_ref[1, :N_NODES, :1] + 1.0
  dis = lax.rsqrt(deg)
  dis_ref[...] = dis
  xw = jnp.dot(x_ref[...], w1_ref[...], preferred_element_type=jnp.float32)
  y1_ref[:N_NODES, :] = xw * dis
  y1_ref[pl.ds(N_NODES, PADR), :] = jnp.zeros((PADR, 32), jnp.float32)


def _stage_mid_body(aggp_ref, y_ref, dis_ref, b_ref, w_ref, ynext_ref):
  agg = (aggp_ref[0, :N_NODES, :] + aggp_ref[1, :N_NODES, :]
         + y_ref[:N_NODES, :])
  dis = dis_ref[...]
  h = jnp.maximum(dis * agg + b_ref[...], 0.0)
  ynext_ref[:N_NODES, :] = dis * jnp.dot(
      h, w_ref[...], preferred_element_type=jnp.float32)
  ynext_ref[pl.ds(N_NODES, PADR), :] = jnp.zeros(
      (PADR, ynext_ref.shape[1]), jnp.float32)


def _stage_fin_body(aggp_ref, y_ref, dis_ref, b_ref, out_ref):
  agg = (aggp_ref[0, :N_NODES, :1] + aggp_ref[1, :N_NODES, :1]
         + y_ref[:N_NODES, :1])
  out_ref[...] = dis_ref[...] * agg + b_ref[...]


def _tc(body, out_shapes, *args):
  return pl.pallas_call(body, out_shape=out_shapes)(*args)


def kernel(x, edge_index, W1, b1, W2, b2, W3, b3):
  n = x.shape[0]
  assert n == N_NODES
  src = edge_index[0].astype(jnp.int32)
  dst = edge_index[1].astype(jnp.int32)
  e = src.shape[0]
  ktot = 2 * NBUF * math.ceil(e / (NW * SUB * NBUF))
  k0 = NBUF * round(ktot * K0_FRAC / NBUF)
  k0 = min(max(k0, NBUF), ktot - NBUF)
  k1 = ktot - k0
  kmax = max(k0, k1)
  ep = NS * ktot * SUB
  pad = ep - e
  ar = jnp.arange(pad, dtype=jnp.int32)
  src = jnp.concatenate([src, N_NODES + ar % PADR])
  dst = jnp.concatenate([dst, (ar * 7) % N_PAD])

  def part(a):
    a0 = a[:NS * k0 * SUB].reshape(NS, k0, SUB)
    a1 = a[NS * k0 * SUB:].reshape(NS, k1, SUB)
    a0 = jnp.pad(a0, ((0, 0), (0, kmax - k0), (0, 0)))
    a1 = jnp.pad(a1, ((0, 0), (0, kmax - k1), (0, 0)))
    return jnp.concatenate([a0, a1], axis=0)

  src3 = part(src)
  dst3 = part(dst)
  k = (k0, k1)

  zeros32 = jnp.zeros((NS, ROWS_PT, 32), jnp.float32)
  zeros8 = jnp.zeros((NS, ROWS_PT, FS), jnp.float32)
  ones8 = jnp.concatenate([
      jnp.ones((N_NODES, FS), jnp.float32),
      jnp.zeros((PADR, FS), jnp.float32)])
  w3p = jnp.pad(W3, ((0, 0), (0, FS - W3.shape[1])))

  agg32 = _make_agg(32, k)
  agg8 = _make_agg(FS, k)

  degp = agg8(ones8, src3, dst3, zeros8).reshape(NC, N_PAD, FS)
  dis, y1 = _tc(
      _stage1_body,
      (jax.ShapeDtypeStruct((N_NODES, 1), jnp.float32),
       jax.ShapeDtypeStruct((N_PAD, 32), jnp.float32)),
      degp, x, W1)

  a1 = agg32(y1, src3, dst3, zeros32).reshape(NC, N_PAD, 32)
  y2 = _tc(_stage_mid_body,
           jax.ShapeDtypeStruct((N_PAD, 32), jnp.float32),
           a1, y1, dis, b1.reshape(1, 32), W2)

  a2 = agg32(y2, src3, dst3, zeros32).reshape(NC, N_PAD, 32)
  y3 = _tc(_stage_mid_body,
           jax.ShapeDtypeStruct((N_PAD, FS), jnp.float32),
           a2, y2, dis, b2.reshape(1, 32), w3p)

  a3 = agg8(y3, src3, dst3, zeros8).reshape(NC, N_PAD, FS)
  out = _tc(_stage_fin_body,
            jax.ShapeDtypeStruct((N_NODES, 1), jnp.float32),
            a3, y3, dis, b3.reshape(1, 1))
  return out[:, 0]


# speedup vs baseline: 2.1331x; 1.1465x over previous
import functools
import math

import jax
import jax.numpy as jnp
from jax import lax
from jax.experimental import pallas as pl
from jax.experimental.pallas import tpu as pltpu
from jax.experimental.pallas import tpu_sc as plsc

N_NODES = 10000
NC = 2
NS = 16
NW = NC * NS
SUB = 128
NBUF = 8
FS = 8
K0_FRAC = 0.5

N_PAD = ((N_NODES + 1 + NS * 8 - 1) // (NS * 8)) * (NS * 8)
ROWS_PT = N_PAD // NS
PADR = N_PAD - N_NODES


def _make_agg(F, K):
  mesh = plsc.VectorSubcoreMesh(
      core_axis_name="c", subcore_axis_name="s", num_cores=NC, num_subcores=NS)

  def body(y_hbm, src_hbm, dst_hbm, zeros_hbm, out_hbm, idx_s, idx_d, rows,
           acc, gsem, ssem, gsem2, ssem2):
    K0, K1 = K
    c = lax.axis_index("c")
    s = lax.axis_index("s")
    wid = c * NS + s
    pltpu.sync_copy(zeros_hbm.at[s], acc.at[pl.ds(s * ROWS_PT, ROWS_PT)])
    pltpu.sync_copy(src_hbm.at[wid], idx_s)
    pltpu.sync_copy(dst_hbm.at[wid], idx_d)
    plsc.subcore_barrier()
    kc = jnp.where(c == 0, K0, K1)

    def fire_g(j0, p, sem):
      for b in range(NBUF):
        pltpu.async_copy(y_hbm.at[idx_s.at[j0 + b]], rows.at[p, b], sem)

    def drain_g(j0, p, sem):
      for b in range(NBUF):
        pltpu.make_async_copy(
            y_hbm.at[idx_s.at[j0 + b]], rows.at[p, b], sem).wait()

    def scatter(j0, p, sem):
      puts = [
          pltpu.async_copy(rows.at[p, b], acc.at[idx_d.at[j0 + b]], sem,
                           add=True)
          for b in range(NBUF)
      ]
      for cp in puts:
        cp.wait()

    fire_g(0, 0, gsem)

    @pl.loop(0, kc, step=2 * NBUF)
    def _chunk_pair(j0):
      jb = j0 + NBUF
      jc = j0 + 2 * NBUF

      @pl.when(jb < kc)
      def _fire_b():
        fire_g(jb, 1, gsem2)

      drain_g(j0, 0, gsem)
      scatter(j0, 0, ssem)

      @pl.when(jc < kc)
      def _fire_c():
        fire_g(jc, 0, gsem)

      @pl.when(jb < kc)
      def _do_b():
        drain_g(jb, 1, gsem2)
        scatter(jb, 1, ssem2)

    plsc.subcore_barrier()
    pltpu.sync_copy(acc.at[pl.ds(s * ROWS_PT, ROWS_PT)], out_hbm.at[c, s])

  return pl.kernel(
      body,
      out_type=jax.ShapeDtypeStruct((NC, NS, ROWS_PT, F), jnp.float32),
      mesh=mesh,
      compiler_params=pltpu.CompilerParams(use_tc_tiling_on_sc=False),
      scratch_types=[
          pltpu.VMEM((max(K), SUB), jnp.int32),
          pltpu.VMEM((max(K), SUB), jnp.int32),
          pltpu.VMEM((2, NBUF, SUB, F), jnp.float32),
          pltpu.VMEM_SHARED((N_PAD, F), jnp.float32),
          pltpu.SemaphoreType.DMA,
          pltpu.SemaphoreType.DMA,
          pltpu.SemaphoreType.DMA,
          pltpu.SemaphoreType.DMA,
      ],
  )


def _stage1_body(degp_ref, x_ref, w1_ref, dis_ref, y1_ref):
  deg = degp_ref[0, :N_NODES, :1] + degp_ref[1, :N_NODES, :1] + 1.0
  dis = lax.rsqrt(deg)
  dis_ref[...] = dis
  xw = jnp.dot(x_ref[...], w1_ref[...], preferred_element_type=jnp.float32)
  y1_ref[:N_NODES, :] = xw * dis
  y1_ref[pl.ds(N_NODES, PADR), :] = jnp.zeros((PADR, 32), jnp.float32)


def _stage_mid_body(aggp_ref, y_ref, dis_ref, b_ref, w_ref, ynext_ref):
  agg = (aggp_ref[0, :N_NODES, :] + aggp_ref[1, :N_NODES, :]
         + y_ref[:N_NODES, :])
  dis = dis_ref[...]
  h = jnp.maximum(dis * agg + b_ref[...], 0.0)
  ynext_ref[:N_NODES, :] = dis * jnp.dot(
      h, w_ref[...], preferred_element_type=jnp.float32)
  ynext_ref[pl.ds(N_NODES, PADR), :] = jnp.zeros(
      (PADR, ynext_ref.shape[1]), jnp.float32)


def _stage_fin_body(aggp_ref, y_ref, dis_ref, b_ref, out_ref):
  agg = (aggp_ref[0, :N_NODES, :1] + aggp_ref[1, :N_NODES, :1]
         + y_ref[:N_NODES, :1])
  out_ref[...] = dis_ref[...] * agg + b_ref[...]


def _tc(body, out_shapes, *args):
  return pl.pallas_call(body, out_shape=out_shapes)(*args)


def kernel(x, edge_index, W1, b1, W2, b2, W3, b3):
  n = x.shape[0]
  assert n == N_NODES
  src = edge_index[0].astype(jnp.int32)
  dst = edge_index[1].astype(jnp.int32)
  e = src.shape[0]
  ktot = 2 * NBUF * math.ceil(e / (NW * SUB * NBUF))
  k0 = NBUF * round(ktot * K0_FRAC / NBUF)
  k0 = min(max(k0, NBUF), ktot - NBUF)
  k1 = ktot - k0
  kmax = max(k0, k1)
  ep = NS * ktot * SUB
  pad = ep - e
  ar = jnp.arange(pad, dtype=jnp.int32)
  src = jnp.concatenate([src, N_NODES + ar % PADR])
  dst = jnp.concatenate([dst, (ar * 7) % N_PAD])

  def part(a):
    a0 = a[:NS * k0 * SUB].reshape(NS, k0, SUB)
    a1 = a[NS * k0 * SUB:].reshape(NS, k1, SUB)
    a0 = jnp.pad(a0, ((0, 0), (0, kmax - k0), (0, 0)))
    a1 = jnp.pad(a1, ((0, 0), (0, kmax - k1), (0, 0)))
    return jnp.concatenate([a0, a1], axis=0)

  src3 = part(src)
  dst3 = part(dst)
  k = (k0, k1)

  zeros32 = jnp.zeros((NS, ROWS_PT, 32), jnp.float32)
  zeros8 = jnp.zeros((NS, ROWS_PT, FS), jnp.float32)
  ones8 = jnp.concatenate([
      jnp.ones((N_NODES, FS), jnp.float32),
      jnp.zeros((PADR, FS), jnp.float32)])
  w3p = jnp.pad(W3, ((0, 0), (0, FS - W3.shape[1])))

  agg32 = _make_agg(32, k)
  agg8 = _make_agg(FS, k)

  degp = agg8(ones8, src3, dst3, zeros8).reshape(NC, N_PAD, FS)
  dis, y1 = _tc(
      _stage1_body,
      (jax.ShapeDtypeStruct((N_NODES, 1), jnp.float32),
       jax.ShapeDtypeStruct((N_PAD, 32), jnp.float32)),
      degp, x, W1)

  a1 = agg32(y1, src3, dst3, zeros32).reshape(NC, N_PAD, 32)
  y2 = _tc(_stage_mid_body,
           jax.ShapeDtypeStruct((N_PAD, 32), jnp.float32),
           a1, y1, dis, b1.reshape(1, 32), W2)

  a2 = agg32(y2, src3, dst3, zeros32).reshape(NC, N_PAD, 32)
  y3 = _tc(_stage_mid_body,
           jax.ShapeDtypeStruct((N_PAD, FS), jnp.float32),
           a2, y2, dis, b2.reshape(1, 32), w3p)

  a3 = agg8(y3, src3, dst3, zeros8).reshape(NC, N_PAD, FS)
  out = _tc(_stage_fin_body,
            jax.ShapeDtypeStruct((N_NODES, 1), jnp.float32),
            a3, y3, dis, b3.reshape(1, 1))
  return out[:, 0]
